# Initial kernel scaffold; baseline (speedup 1.0000x reference)
import jax
import jax.numpy as jnp
from jax.experimental import pallas as pl


def kernel(x, edge_index, params):
    raise NotImplementedError("write your pallas kernel here")


# speedup vs baseline: 5.0699x; 5.0699x over previous
import functools

import jax
import jax.numpy as jnp
from jax import lax
from jax.experimental import pallas as pl
from jax.experimental.pallas import tpu as pltpu
from jax.experimental.pallas import tpu_sc as plsc

N = 10000
E = 320000
M = 128
H = 16

_NC = 2
_NS = 16
_EPT = E // (_NC * _NS)
_C = 80

bf16 = jnp.bfloat16
f32 = jnp.float32


def _make_sc_edge(dout):
  mesh = plsc.VectorSubcoreMesh(core_axis_name="c", subcore_axis_name="s")

  @functools.partial(
      pl.kernel, mesh=mesh,
      out_type=jax.ShapeDtypeStruct((_NC, N, dout), f32),
      compiler_params=pltpu.CompilerParams(use_tc_tiling_on_sc=False),
      scratch_types=[
          pltpu.VMEM((_C,), jnp.int32),
          pltpu.VMEM((_C,), jnp.int32),
          pltpu.VMEM((_C, dout), f32),
          pltpu.VMEM((_C, dout), f32),
          pltpu.VMEM_SHARED((N, dout), f32),
          pltpu.SemaphoreType.DMA,
          pltpu.SemaphoreType.DMA,
      ],
  )
  def sc_edge(a_hbm, b_hbm, src_hbm, dst_hbm, zero_hbm, out_hbm,
              idx_s, idx_d, arows, brows, s_sh, sem1, sem2):
    c = lax.axis_index("c")
    s = lax.axis_index("s")

    @pl.when(s == 0)
    def _():
      pltpu.sync_copy(zero_hbm, s_sh)

    plsc.subcore_barrier()
    base = (c * _NS + s) * _EPT

    def chunk(j, carry):
      e0 = base + j * _C
      pltpu.sync_copy(src_hbm.at[pl.ds(e0, _C)], idx_s)
      pltpu.sync_copy(dst_hbm.at[pl.ds(e0, _C)], idx_d)
      cp_b = pltpu.async_copy(b_hbm.at[idx_s], brows, sem1)
      cp_a = pltpu.async_copy(a_hbm.at[idx_d], arows, sem2)
      cp_b.wait()
      cp_a.wait()

      def erow(e, carry2):
        for k in range(dout // 16):
          av = arows[e, pl.ds(k * 16, 16)]
          bv = brows[e, pl.ds(k * 16, 16)]
          m = jnp.maximum(av + bv, 0.0)
          u = lax.bitcast_convert_type(m, jnp.uint32)
          u = (u + jnp.uint32(0x7FFF) + ((u >> 16) & jnp.uint32(1))) & jnp.uint32(0xFFFF0000)
          arows[e, pl.ds(k * 16, 16)] = lax.bitcast_convert_type(u, f32)
        return carry2

      lax.fori_loop(0, _C, erow, 0)
      pltpu.sync_copy(arows, s_sh.at[idx_d], add=True)
      return carry

    lax.fori_loop(0, _EPT // _C, chunk, 0)
    plsc.subcore_barrier()

    @pl.when(s == 0)
    def _():
      pltpu.sync_copy(s_sh, out_hbm.at[c])

  return sc_edge


def _make_sc_deg():
  mesh = plsc.VectorSubcoreMesh(core_axis_name="c", subcore_axis_name="s")

  @functools.partial(
      pl.kernel, mesh=mesh,
      out_type=jax.ShapeDtypeStruct((_NC, N), f32),
      compiler_params=pltpu.CompilerParams(use_tc_tiling_on_sc=False),
      scratch_types=[
          pltpu.VMEM((_C,), jnp.int32),
          pltpu.VMEM((_C,), f32),
          pltpu.VMEM_SHARED((N,), f32),
      ],
  )
  def sc_deg(dst_hbm, zero_hbm, out_hbm, idx_d, ones_v, d_sh):
    c = lax.axis_index("c")
    s = lax.axis_index("s")
    for t in range(_C // 16):
      ones_v[pl.ds(t * 16, 16)] = jnp.ones((16,), f32)

    @pl.when(s == 0)
    def _():
      pltpu.sync_copy(zero_hbm, d_sh)

    plsc.subcore_barrier()
    base = (c * _NS + s) * _EPT

    def chunk(j, carry):
      e0 = base + j * _C
      pltpu.sync_copy(dst_hbm.at[pl.ds(e0, _C)], idx_d)
      pltpu.sync_copy(ones_v, d_sh.at[idx_d], add=True)
      return carry

    lax.fori_loop(0, _EPT // _C, chunk, 0)
    plsc.subcore_barrier()

    @pl.when(s == 0)
    def _():
      pltpu.sync_copy(d_sh, out_hbm.at[c])

  return sc_deg


def _dx(a, w):
  return jnp.dot(a.astype(bf16), w.astype(bf16), preferred_element_type=f32)


def _tc_in(x2, wt, ws, mb1):
  dout = wt.shape[1]

  def body(x_ref, wt_ref, ws_ref, mb1_ref, a_ref, b_ref):
    h = x_ref[...]
    a_ref[...] = _dx(h, wt_ref[...]) + mb1_ref[...]
    b_ref[...] = _dx(h, ws_ref[...])

  return pl.pallas_call(
      body,
      out_shape=[jax.ShapeDtypeStruct((N, dout), f32),
                 jax.ShapeDtypeStruct((N, dout), f32)],
  )(x2, wt, ws, mb1)


def _node_update(h_ref, s2_ref, deg2_ref, mw2_ref, mb2_ref, uw1_ref,
                 ub1_ref, uw2_ref, ub2_ref, bng_ref, bnb_ref, resw_ref,
                 resb_ref):
  h_ = h_ref[...]
  s_ = s2_ref[0] + s2_ref[1]
  deg = deg2_ref[0] + deg2_ref[1]
  wb = mw2_ref[...].astype(bf16)
  p1_ = s_.astype(bf16)
  r1 = s_ - p1_.astype(f32)
  p2_ = r1.astype(bf16)
  p3_ = (r1 - p2_.astype(f32)).astype(bf16)
  agg = (jnp.dot(p1_, wb, preferred_element_type=f32)
         + jnp.dot(p2_, wb, preferred_element_type=f32)
         + jnp.dot(p3_, wb, preferred_element_type=f32)
         + deg * mb2_ref[...])
  u_in = jnp.concatenate([h_, agg], axis=-1)
  u = jnp.maximum(_dx(u_in, uw1_ref[...]) + ub1_ref[...], 0.0)
  u = _dx(u, uw2_ref[...]) + ub2_ref[...]
  mu = jnp.mean(u, axis=0, keepdims=True)
  var = jnp.mean((u - mu) ** 2, axis=0, keepdims=True)
  u = (u - mu) / jnp.sqrt(var + 1e-5) * bng_ref[...] + bnb_ref[...]
  return jnp.maximum(u + _dx(h_, resw_ref[...]) + resb_ref[...], 0.0)


def _tc_mid(h, s2, deg2, p, wt_n, ws_n, mb1_n):
  din = h.shape[1]
  dout = p['mW2'].shape[1]
  dnext = wt_n.shape[1]

  def body(h_ref, s2_ref, deg2_ref, mw2_ref, mb2_ref, uw1_ref,
           ub1_ref, uw2_ref, ub2_ref, bng_ref, bnb_ref, resw_ref, resb_ref,
           wtn_ref, wsn_ref, mb1n_ref, hn_ref, a_ref, b_ref):
    hn = _node_update(h_ref, s2_ref, deg2_ref, mw2_ref, mb2_ref, uw1_ref,
                      ub1_ref, uw2_ref, ub2_ref, bng_ref, bnb_ref, resw_ref,
                      resb_ref)
    hn_ref[...] = hn
    a_ref[...] = _dx(hn, wtn_ref[...]) + mb1n_ref[...]
    b_ref[...] = _dx(hn, wsn_ref[...])

  return pl.pallas_call(
      body,
      out_shape=[jax.ShapeDtypeStruct((N, dout), f32),
                 jax.ShapeDtypeStruct((N, dnext), f32),
                 jax.ShapeDtypeStruct((N, dnext), f32)],
  )(h, s2, deg2,
    p['mW2'], p['mb2'].reshape(1, -1),
    p['uW1'], p['ub1'].reshape(1, -1),
    p['uW2'], p['ub2'].reshape(1, -1),
    p['bn_g'].reshape(1, -1), p['bn_b'].reshape(1, -1),
    p['resW'], p['resb'].reshape(1, -1),
    wt_n, ws_n, mb1_n)


def _tc_final(h, s2, deg2, p, ln_g, ln_b):
  dout = p['mW2'].shape[1]

  def body(h_ref, s2_ref, deg2_ref, mw2_ref, mb2_ref, uw1_ref,
           ub1_ref, uw2_ref, ub2_ref, bng_ref, bnb_ref, resw_ref, resb_ref,
           lng_ref, lnb_ref, out_ref):
    hn = _node_update(h_ref, s2_ref, deg2_ref, mw2_ref, mb2_ref, uw1_ref,
                      ub1_ref, uw2_ref, ub2_ref, bng_ref, bnb_ref, resw_ref,
                      resb_ref)
    mu2 = jnp.mean(hn, axis=1, keepdims=True)
    var2 = jnp.mean((hn - mu2) ** 2, axis=1, keepdims=True)
    out_ref[...] = (hn - mu2) / jnp.sqrt(var2 + 1e-5) * lng_ref[...] + lnb_ref[...]

  return pl.pallas_call(
      body,
      out_shape=jax.ShapeDtypeStruct((N, dout), f32),
  )(h, s2, deg2,
    p['mW2'], p['mb2'].reshape(1, -1),
    p['uW1'], p['ub1'].reshape(1, -1),
    p['uW2'], p['ub2'].reshape(1, -1),
    p['bn_g'].reshape(1, -1), p['bn_b'].reshape(1, -1),
    p['resW'], p['resb'].reshape(1, -1),
    ln_g.reshape(1, -1), ln_b.reshape(1, -1))


def kernel(x, edge_index, params):
  x2 = x[0, :, :, 0]
  src = edge_index[0]
  dst = edge_index[1]

  deg2 = _make_sc_deg()(dst, jnp.zeros((N,), f32))
  deg2 = deg2.reshape(_NC, N, 1)

  p1 = params['mpnn1']
  a, b = _tc_in(x2, p1['mW1'][:M], p1['mW1'][M:], p1['mb1'].reshape(1, -1))

  h = x2
  out = None
  for i in range(1, 6):
    p = params['mpnn%d' % i]
    dout = p['mW2'].shape[1]
    s2 = _make_sc_edge(dout)(a, b, src, dst, jnp.zeros((N, dout), f32))
    if i < 5:
      pn = params['mpnn%d' % (i + 1)]
      h, a, b = _tc_mid(h, s2, deg2, p,
                        pn['mW1'][:dout], pn['mW1'][dout:],
                        pn['mb1'].reshape(1, -1))
    else:
      out = _tc_final(h, s2, deg2, p, params['ln_g'], params['ln_b'])
  return out.reshape(1, -1)


# speedup vs baseline: 6.8052x; 1.3423x over previous
import functools

import jax
import jax.numpy as jnp
from jax import lax
from jax.experimental import pallas as pl
from jax.experimental.pallas import tpu as pltpu
from jax.experimental.pallas import tpu_sc as plsc

N = 10000
E = 320000
M = 128
H = 16

_NC = 2
_NS = 16
_EPT = E // (_NC * _NS)
_C = 80

bf16 = jnp.bfloat16
f32 = jnp.float32


def _make_sc_edge(dout):
  mesh = plsc.VectorSubcoreMesh(core_axis_name="c", subcore_axis_name="s")

  nchunk = _EPT // _C

  @functools.partial(
      pl.kernel, mesh=mesh,
      out_type=jax.ShapeDtypeStruct((_NC, N, dout), f32),
      compiler_params=pltpu.CompilerParams(use_tc_tiling_on_sc=False),
      scratch_types=[
          pltpu.VMEM((_C,), jnp.int32),
          pltpu.VMEM((_C,), jnp.int32),
          pltpu.VMEM((_C,), jnp.int32),
          pltpu.VMEM((_C,), jnp.int32),
          pltpu.VMEM((_C, dout), f32),
          pltpu.VMEM((_C, dout), f32),
          pltpu.VMEM((_C, dout), f32),
          pltpu.VMEM((_C, dout), f32),
          pltpu.VMEM_SHARED((N, dout), f32),
          pltpu.SemaphoreType.DMA,
          pltpu.SemaphoreType.DMA,
          pltpu.SemaphoreType.DMA,
          pltpu.SemaphoreType.DMA,
      ],
  )
  def sc_edge(a_hbm, b_hbm, src_hbm, dst_hbm, zero_hbm, out_hbm,
              idx_s0, idx_s1, idx_d0, idx_d1, arows0, arows1, brows0, brows1,
              s_sh, sa0, sa1, sb0, sb1):
    c = lax.axis_index("c")
    s = lax.axis_index("s")
    idx_s = (idx_s0, idx_s1)
    idx_d = (idx_d0, idx_d1)
    arows = (arows0, arows1)
    brows = (brows0, brows1)
    sa = (sa0, sa1)
    sb = (sb0, sb1)

    @pl.when(s == 0)
    def _():
      pltpu.sync_copy(zero_hbm, s_sh)

    plsc.subcore_barrier()
    base = (c * _NS + s) * _EPT

    def fetch(j, p):
      e0 = base + j * _C
      pltpu.sync_copy(src_hbm.at[pl.ds(e0, _C)], idx_s[p])
      pltpu.sync_copy(dst_hbm.at[pl.ds(e0, _C)], idx_d[p])
      pltpu.async_copy(b_hbm.at[idx_s[p]], brows[p], sb[p])
      pltpu.async_copy(a_hbm.at[idx_d[p]], arows[p], sa[p])

    def work(j, p):
      pltpu.make_async_copy(b_hbm.at[idx_s[p]], brows[p], sb[p]).wait()
      pltpu.make_async_copy(a_hbm.at[idx_d[p]], arows[p], sa[p]).wait()

      def erow(e, carry2):
        for k in range(dout // 16):
          av = arows[p][e, pl.ds(k * 16, 16)]
          bv = brows[p][e, pl.ds(k * 16, 16)]
          m = jnp.maximum(av + bv, 0.0)
          u = lax.bitcast_convert_type(m, jnp.uint32)
          u = (u + jnp.uint32(0x7FFF) + ((u >> 16) & jnp.uint32(1))) & jnp.uint32(0xFFFF0000)
          arows[p][e, pl.ds(k * 16, 16)] = lax.bitcast_convert_type(u, f32)
        return carry2

      lax.fori_loop(0, _C, erow, 0)
      pltpu.sync_copy(arows[p], s_sh.at[idx_d[p]], add=True)

    fetch(0, 0)

    def dbl(t, carry):
      for p in range(2):
        j = 2 * t + p

        @pl.when(j < nchunk)
        def _():
          @pl.when(j + 1 < nchunk)
          def _():
            fetch(j + 1, 1 - p)

          work(j, p)

      return carry

    lax.fori_loop(0, (nchunk + 1) // 2, dbl, 0)
    plsc.subcore_barrier()

    @pl.when(s == 0)
    def _():
      pltpu.sync_copy(s_sh, out_hbm.at[c])

  return sc_edge


def _make_sc_deg():
  mesh = plsc.VectorSubcoreMesh(core_axis_name="c", subcore_axis_name="s")

  @functools.partial(
      pl.kernel, mesh=mesh,
      out_type=jax.ShapeDtypeStruct((_NC, N), f32),
      compiler_params=pltpu.CompilerParams(use_tc_tiling_on_sc=False),
      scratch_types=[
          pltpu.VMEM((_C,), jnp.int32),
          pltpu.VMEM((_C,), f32),
          pltpu.VMEM_SHARED((N,), f32),
      ],
  )
  def sc_deg(dst_hbm, zero_hbm, out_hbm, idx_d, ones_v, d_sh):
    c = lax.axis_index("c")
    s = lax.axis_index("s")
    for t in range(_C // 16):
      ones_v[pl.ds(t * 16, 16)] = jnp.ones((16,), f32)

    @pl.when(s == 0)
    def _():
      pltpu.sync_copy(zero_hbm, d_sh)

    plsc.subcore_barrier()
    base = (c * _NS + s) * _EPT

    def chunk(j, carry):
      e0 = base + j * _C
      pltpu.sync_copy(dst_hbm.at[pl.ds(e0, _C)], idx_d)
      pltpu.sync_copy(ones_v, d_sh.at[idx_d], add=True)
      return carry

    lax.fori_loop(0, _EPT // _C, chunk, 0)
    plsc.subcore_barrier()

    @pl.when(s == 0)
    def _():
      pltpu.sync_copy(d_sh, out_hbm.at[c])

  return sc_deg


def _dx(a, w):
  return jnp.dot(a.astype(bf16), w.astype(bf16), preferred_element_type=f32)


def _tc_in(x2, wt, ws, mb1):
  dout = wt.shape[1]

  def body(x_ref, wt_ref, ws_ref, mb1_ref, a_ref, b_ref):
    h = x_ref[...]
    a_ref[...] = _dx(h, wt_ref[...]) + mb1_ref[...]
    b_ref[...] = _dx(h, ws_ref[...])

  return pl.pallas_call(
      body,
      out_shape=[jax.ShapeDtypeStruct((N, dout), f32),
                 jax.ShapeDtypeStruct((N, dout), f32)],
  )(x2, wt, ws, mb1)


def _node_update(h_ref, s2_ref, deg2_ref, mw2_ref, mb2_ref, uw1_ref,
                 ub1_ref, uw2_ref, ub2_ref, bng_ref, bnb_ref, resw_ref,
                 resb_ref):
  h_ = h_ref[...]
  s_ = s2_ref[0] + s2_ref[1]
  deg = deg2_ref[0] + deg2_ref[1]
  wb = mw2_ref[...].astype(bf16)
  p1_ = s_.astype(bf16)
  r1 = s_ - p1_.astype(f32)
  p2_ = r1.astype(bf16)
  p3_ = (r1 - p2_.astype(f32)).astype(bf16)
  agg = (jnp.dot(p1_, wb, preferred_element_type=f32)
         + jnp.dot(p2_, wb, preferred_element_type=f32)
         + jnp.dot(p3_, wb, preferred_element_type=f32)
         + deg * mb2_ref[...])
  u_in = jnp.concatenate([h_, agg], axis=-1)
  u = jnp.maximum(_dx(u_in, uw1_ref[...]) + ub1_ref[...], 0.0)
  u = _dx(u, uw2_ref[...]) + ub2_ref[...]
  mu = jnp.mean(u, axis=0, keepdims=True)
  var = jnp.mean((u - mu) ** 2, axis=0, keepdims=True)
  u = (u - mu) / jnp.sqrt(var + 1e-5) * bng_ref[...] + bnb_ref[...]
  return jnp.maximum(u + _dx(h_, resw_ref[...]) + resb_ref[...], 0.0)


def _tc_mid(h, s2, deg2, p, wt_n, ws_n, mb1_n):
  din = h.shape[1]
  dout = p['mW2'].shape[1]
  dnext = wt_n.shape[1]

  def body(h_ref, s2_ref, deg2_ref, mw2_ref, mb2_ref, uw1_ref,
           ub1_ref, uw2_ref, ub2_ref, bng_ref, bnb_ref, resw_ref, resb_ref,
           wtn_ref, wsn_ref, mb1n_ref, hn_ref, a_ref, b_ref):
    hn = _node_update(h_ref, s2_ref, deg2_ref, mw2_ref, mb2_ref, uw1_ref,
                      ub1_ref, uw2_ref, ub2_ref, bng_ref, bnb_ref, resw_ref,
                      resb_ref)
    hn_ref[...] = hn
    a_ref[...] = _dx(hn, wtn_ref[...]) + mb1n_ref[...]
    b_ref[...] = _dx(hn, wsn_ref[...])

  return pl.pallas_call(
      body,
      out_shape=[jax.ShapeDtypeStruct((N, dout), f32),
                 jax.ShapeDtypeStruct((N, dnext), f32),
                 jax.ShapeDtypeStruct((N, dnext), f32)],
  )(h, s2, deg2,
    p['mW2'], p['mb2'].reshape(1, -1),
    p['uW1'], p['ub1'].reshape(1, -1),
    p['uW2'], p['ub2'].reshape(1, -1),
    p['bn_g'].reshape(1, -1), p['bn_b'].reshape(1, -1),
    p['resW'], p['resb'].reshape(1, -1),
    wt_n, ws_n, mb1_n)


def _tc_final(h, s2, deg2, p, ln_g, ln_b):
  dout = p['mW2'].shape[1]

  def body(h_ref, s2_ref, deg2_ref, mw2_ref, mb2_ref, uw1_ref,
           ub1_ref, uw2_ref, ub2_ref, bng_ref, bnb_ref, resw_ref, resb_ref,
           lng_ref, lnb_ref, out_ref):
    hn = _node_update(h_ref, s2_ref, deg2_ref, mw2_ref, mb2_ref, uw1_ref,
                      ub1_ref, uw2_ref, ub2_ref, bng_ref, bnb_ref, resw_ref,
                      resb_ref)
    mu2 = jnp.mean(hn, axis=1, keepdims=True)
    var2 = jnp.mean((hn - mu2) ** 2, axis=1, keepdims=True)
    out_ref[...] = (hn - mu2) / jnp.sqrt(var2 + 1e-5) * lng_ref[...] + lnb_ref[...]

  return pl.pallas_call(
      body,
      out_shape=jax.ShapeDtypeStruct((N, dout), f32),
  )(h, s2, deg2,
    p['mW2'], p['mb2'].reshape(1, -1),
    p['uW1'], p['ub1'].reshape(1, -1),
    p['uW2'], p['ub2'].reshape(1, -1),
    p['bn_g'].reshape(1, -1), p['bn_b'].reshape(1, -1),
    p['resW'], p['resb'].reshape(1, -1),
    ln_g.reshape(1, -1), ln_b.reshape(1, -1))


def kernel(x, edge_index, params):
  x2 = x[0, :, :, 0]
  src = edge_index[0]
  dst = edge_index[1]

  deg2 = _make_sc_deg()(dst, jnp.zeros((N,), f32))
  deg2 = deg2.reshape(_NC, N, 1)

  p1 = params['mpnn1']
  a, b = _tc_in(x2, p1['mW1'][:M], p1['mW1'][M:], p1['mb1'].reshape(1, -1))

  h = x2
  out = None
  for i in range(1, 6):
    p = params['mpnn%d' % i]
    dout = p['mW2'].shape[1]
    s2 = _make_sc_edge(dout)(a, b, src, dst, jnp.zeros((N, dout), f32))
    if i < 5:
      pn = params['mpnn%d' % (i + 1)]
      h, a, b = _tc_mid(h, s2, deg2, p,
                        pn['mW1'][:dout], pn['mW1'][dout:],
                        pn['mb1'].reshape(1, -1))
    else:
      out = _tc_final(h, s2, deg2, p, params['ln_g'], params['ln_b'])
  return out.reshape(1, -1)
